# baseline (speedup 1.0000x reference)
import math
from functools import partial

import jax
import jax.numpy as jnp
from jax import lax
from jax.experimental import pallas as pl
from jax.experimental.pallas import tpu as pltpu

_NB = 2


def _stack_kernel(L, H, NB,
                  x_ref, wlin_ref, blin_ref,
                  wqkv_ref, bqkv_ref, wout_ref, bout_ref, wffn_ref,
                  o_ref, p_sc, qkv_sc):
    C, S = x_ref.shape[1], x_ref.shape[2]
    D = C // H
    bf16 = jnp.bfloat16

    xt = jnp.concatenate([x_ref[i].T for i in range(NB)], axis=0)
    p_sc[...] = (xt
                 + jnp.dot(xt.astype(bf16), wlin_ref[...],
                           preferred_element_type=jnp.float32)
                 + blin_ref[...])

    trans_b = (((1,), (1,)), ((), ()))
    ones_col = jnp.ones((S, D), bf16)

    def layer(l, carry):
        p = p_sc[...]
        qkv_sc[...] = (jnp.dot(p.astype(bf16), wqkv_ref[l],
                               preferred_element_type=jnp.float32)
                       + bqkv_ref[l]).astype(bf16)

        rows = []
        for i in range(NB):
            r0, r1 = i * S, (i + 1) * S
            heads = []
            for h in range(H):
                q = qkv_sc[r0:r1, h * D:(h + 1) * D]
                k = qkv_sc[r0:r1, C + h * D:C + (h + 1) * D]
                v = qkv_sc[r0:r1, 2 * C + h * D:2 * C + (h + 1) * D]
                s = lax.dot_general(q, k, trans_b,
                                    preferred_element_type=jnp.float32)
                e = jnp.exp2(jnp.minimum(s, 115.0)).astype(bf16)
                o = jnp.dot(e, jnp.concatenate([v, ones_col], axis=1),
                            preferred_element_type=jnp.float32)
                heads.append(o[:, :D]
                             * pl.reciprocal(o[:, D:2 * D], approx=True))
            rows.append(jnp.concatenate(heads, axis=1))
        attn = jnp.concatenate(rows, axis=0).astype(bf16)

        x1 = (jnp.dot(attn, wout_ref[l], preferred_element_type=jnp.float32)
              + bout_ref[l] + p)
        p_sc[...] = x1 + jnp.dot(x1.astype(bf16), wffn_ref[l],
                                 preferred_element_type=jnp.float32)
        return carry

    lax.fori_loop(0, L, layer, 0)
    for i in range(NB):
        o_ref[i] = p_sc[i * S:(i + 1) * S, :].T


def kernel(x, linear_w, linear_b, lq, lk, lv, lin_proj_w, lin_proj_b,
           lout_w, lout_b, lfc1, lfc2):
    num_heads = 8
    B, C, W, Hs = x.shape
    S = W * Hs
    L = lq.shape[0]
    scale = math.log2(math.e) / math.sqrt(C // num_heads)

    wq_eff = jnp.einsum("lij,ljk->lik", lin_proj_w[:, 0:C], lq) * scale
    wk_eff = jnp.einsum("lij,ljk->lik", lin_proj_w[:, C:2 * C], lk)
    wv_eff = jnp.einsum("lij,ljk->lik", lin_proj_w[:, 2 * C:3 * C], lv)
    wqkv_t = jnp.concatenate([wq_eff.transpose(0, 2, 1),
                              wk_eff.transpose(0, 2, 1),
                              wv_eff.transpose(0, 2, 1)], axis=2)
    bqkv = jnp.concatenate([lin_proj_b[:, 0] * scale,
                            lin_proj_b[:, 1],
                            lin_proj_b[:, 2]], axis=-1)[:, None, :]
    wout_t = lout_w.transpose(0, 2, 1)
    wffn_t = jnp.einsum("lij,ljk->lik", lfc2, lfc1).transpose(0, 2, 1)

    bf16 = jnp.bfloat16
    wlin_t = linear_w.T.astype(bf16)
    wqkv_t = wqkv_t.astype(bf16)
    wout_t = wout_t.astype(bf16)
    wffn_t = wffn_t.astype(bf16)

    xr = x.reshape(B, C, S)
    NB = _NB

    G = B // NB
    out = pl.pallas_call(
        partial(_stack_kernel, L, num_heads, NB),
        out_shape=jax.ShapeDtypeStruct((B, C, S), x.dtype),
        grid=(2, G // 2),
        in_specs=[
            pl.BlockSpec((NB, C, S), lambda c, b: (c * (G // 2) + b, 0, 0)),
            pl.BlockSpec((C, C), lambda c, b: (0, 0)),
            pl.BlockSpec((1, C), lambda c, b: (0, 0)),
            pl.BlockSpec((L, C, 3 * C), lambda c, b: (0, 0, 0)),
            pl.BlockSpec((L, 1, 3 * C), lambda c, b: (0, 0, 0)),
            pl.BlockSpec((L, C, C), lambda c, b: (0, 0, 0)),
            pl.BlockSpec((L, 1, C), lambda c, b: (0, 0, 0)),
            pl.BlockSpec((L, C, C), lambda c, b: (0, 0, 0)),
        ],
        out_specs=pl.BlockSpec((NB, C, S),
                               lambda c, b: (c * (G // 2) + b, 0, 0)),
        scratch_shapes=[
            pltpu.VMEM((NB * S, C), jnp.float32),
            pltpu.VMEM((NB * S, 3 * C), jnp.bfloat16),
        ],
        compiler_params=pltpu.CompilerParams(
            dimension_semantics=("parallel", "arbitrary")),
    )(xr, wlin_t, linear_b, wqkv_t, bqkv, wout_t, lout_b, wffn_t)

    return out.reshape(B, C, W, Hs)


# speedup vs baseline: 1.2147x; 1.2147x over previous
import math
from functools import partial

import jax
import jax.numpy as jnp
from jax import lax
from jax.experimental import pallas as pl
from jax.experimental.pallas import tpu as pltpu

_NB = 4


def _stack_kernel(L, H, NB,
                  x_ref, wlin_ref, blin_ref,
                  wqkv_ref, bqkv_ref, wout_ref, bout_ref, wffn_ref,
                  o_ref, p_sc, qkv_sc):
    C, S = x_ref.shape[1], x_ref.shape[2]
    D = C // H
    bf16 = jnp.bfloat16

    xt = jnp.concatenate([x_ref[i].T for i in range(NB)], axis=0)
    p_sc[...] = (xt
                 + jnp.dot(xt.astype(bf16), wlin_ref[...],
                           preferred_element_type=jnp.float32)
                 + blin_ref[...])

    trans_b = (((1,), (1,)), ((), ()))
    ones_col = jnp.ones((S, D), bf16)

    def layer(l, carry):
        p = p_sc[...]
        qkv_sc[...] = (jnp.dot(p.astype(bf16), wqkv_ref[l],
                               preferred_element_type=jnp.float32)
                       + bqkv_ref[l]).astype(bf16)

        rows = []
        for i in range(NB):
            r0, r1 = i * S, (i + 1) * S
            heads = []
            for h in range(H):
                q = qkv_sc[r0:r1, h * D:(h + 1) * D]
                k = qkv_sc[r0:r1, C + h * D:C + (h + 1) * D]
                v = qkv_sc[r0:r1, 2 * C + h * D:2 * C + (h + 1) * D]
                s = lax.dot_general(q, k, trans_b,
                                    preferred_element_type=jnp.float32)
                e = jnp.exp2(jnp.minimum(s, 115.0)).astype(bf16)
                o = jnp.dot(e, jnp.concatenate([v, ones_col], axis=1),
                            preferred_element_type=jnp.float32)
                heads.append(o[:, :D]
                             * pl.reciprocal(o[:, D:2 * D], approx=True))
            rows.append(jnp.concatenate(heads, axis=1))
        attn = jnp.concatenate(rows, axis=0).astype(bf16)

        x1 = (jnp.dot(attn, wout_ref[l], preferred_element_type=jnp.float32)
              + bout_ref[l] + p)
        p_sc[...] = x1 + jnp.dot(x1.astype(bf16), wffn_ref[l],
                                 preferred_element_type=jnp.float32)
        return carry

    lax.fori_loop(0, L, layer, 0)
    for i in range(NB):
        o_ref[i] = p_sc[i * S:(i + 1) * S, :].T


def kernel(x, linear_w, linear_b, lq, lk, lv, lin_proj_w, lin_proj_b,
           lout_w, lout_b, lfc1, lfc2):
    num_heads = 8
    B, C, W, Hs = x.shape
    S = W * Hs
    L = lq.shape[0]
    scale = math.log2(math.e) / math.sqrt(C // num_heads)

    wq_eff = jnp.einsum("lij,ljk->lik", lin_proj_w[:, 0:C], lq) * scale
    wk_eff = jnp.einsum("lij,ljk->lik", lin_proj_w[:, C:2 * C], lk)
    wv_eff = jnp.einsum("lij,ljk->lik", lin_proj_w[:, 2 * C:3 * C], lv)
    wqkv_t = jnp.concatenate([wq_eff.transpose(0, 2, 1),
                              wk_eff.transpose(0, 2, 1),
                              wv_eff.transpose(0, 2, 1)], axis=2)
    bqkv = jnp.concatenate([lin_proj_b[:, 0] * scale,
                            lin_proj_b[:, 1],
                            lin_proj_b[:, 2]], axis=-1)[:, None, :]
    wout_t = lout_w.transpose(0, 2, 1)
    wffn_t = jnp.einsum("lij,ljk->lik", lfc2, lfc1).transpose(0, 2, 1)

    bf16 = jnp.bfloat16
    wlin_t = linear_w.T.astype(bf16)
    wqkv_t = wqkv_t.astype(bf16)
    wout_t = wout_t.astype(bf16)
    wffn_t = wffn_t.astype(bf16)

    xr = x.reshape(B, C, S)
    NB = _NB

    G = B // NB
    out = pl.pallas_call(
        partial(_stack_kernel, L, num_heads, NB),
        out_shape=jax.ShapeDtypeStruct((B, C, S), x.dtype),
        grid=(2, G // 2),
        in_specs=[
            pl.BlockSpec((NB, C, S), lambda c, b: (c * (G // 2) + b, 0, 0)),
            pl.BlockSpec((C, C), lambda c, b: (0, 0)),
            pl.BlockSpec((1, C), lambda c, b: (0, 0)),
            pl.BlockSpec((L, C, 3 * C), lambda c, b: (0, 0, 0)),
            pl.BlockSpec((L, 1, 3 * C), lambda c, b: (0, 0, 0)),
            pl.BlockSpec((L, C, C), lambda c, b: (0, 0, 0)),
            pl.BlockSpec((L, 1, C), lambda c, b: (0, 0, 0)),
            pl.BlockSpec((L, C, C), lambda c, b: (0, 0, 0)),
        ],
        out_specs=pl.BlockSpec((NB, C, S),
                               lambda c, b: (c * (G // 2) + b, 0, 0)),
        scratch_shapes=[
            pltpu.VMEM((NB * S, C), jnp.float32),
            pltpu.VMEM((NB * S, 3 * C), jnp.bfloat16),
        ],
        compiler_params=pltpu.CompilerParams(
            dimension_semantics=("parallel", "arbitrary")),
    )(xr, wlin_t, linear_b, wqkv_t, bqkv, wout_t, lout_b, wffn_t)

    return out.reshape(B, C, W, Hs)


# speedup vs baseline: 1.3284x; 1.0936x over previous
import math
from functools import partial

import jax
import jax.numpy as jnp
from jax import lax
from jax.experimental import pallas as pl
from jax.experimental.pallas import tpu as pltpu

_NB = 8


def _stack_kernel(L, H, NB,
                  x_ref, wlin_ref, blin_ref,
                  wqkv_ref, bqkv_ref, wout_ref, bout_ref, wffn_ref,
                  o_ref, p_sc, qkv_sc):
    C, S = x_ref.shape[1], x_ref.shape[2]
    D = C // H
    bf16 = jnp.bfloat16

    xt = jnp.concatenate([x_ref[i].T for i in range(NB)], axis=0)
    p_sc[...] = (xt
                 + jnp.dot(xt.astype(bf16), wlin_ref[...],
                           preferred_element_type=jnp.float32)
                 + blin_ref[...])

    trans_b = (((1,), (1,)), ((), ()))
    ones_col = jnp.ones((S, D), bf16)

    def layer(l, carry):
        p = p_sc[...]
        qkv_sc[...] = (jnp.dot(p.astype(bf16), wqkv_ref[l],
                               preferred_element_type=jnp.float32)
                       + bqkv_ref[l]).astype(bf16)

        rows = []
        for i in range(NB):
            r0, r1 = i * S, (i + 1) * S
            heads = []
            for h in range(H):
                q = qkv_sc[r0:r1, h * D:(h + 1) * D]
                k = qkv_sc[r0:r1, C + h * D:C + (h + 1) * D]
                v = qkv_sc[r0:r1, 2 * C + h * D:2 * C + (h + 1) * D]
                s = lax.dot_general(q, k, trans_b,
                                    preferred_element_type=jnp.float32)
                e = jnp.exp2(jnp.minimum(s, 115.0)).astype(bf16)
                o = jnp.dot(e, jnp.concatenate([v, ones_col], axis=1),
                            preferred_element_type=jnp.float32)
                heads.append(o[:, :D]
                             * pl.reciprocal(o[:, D:2 * D], approx=True))
            rows.append(jnp.concatenate(heads, axis=1))
        attn = jnp.concatenate(rows, axis=0).astype(bf16)

        x1 = (jnp.dot(attn, wout_ref[l], preferred_element_type=jnp.float32)
              + bout_ref[l] + p)
        p_sc[...] = x1 + jnp.dot(x1.astype(bf16), wffn_ref[l],
                                 preferred_element_type=jnp.float32)
        return carry

    lax.fori_loop(0, L, layer, 0)
    for i in range(NB):
        o_ref[i] = p_sc[i * S:(i + 1) * S, :].T


def kernel(x, linear_w, linear_b, lq, lk, lv, lin_proj_w, lin_proj_b,
           lout_w, lout_b, lfc1, lfc2):
    num_heads = 8
    B, C, W, Hs = x.shape
    S = W * Hs
    L = lq.shape[0]
    scale = math.log2(math.e) / math.sqrt(C // num_heads)

    wq_eff = jnp.einsum("lij,ljk->lik", lin_proj_w[:, 0:C], lq) * scale
    wk_eff = jnp.einsum("lij,ljk->lik", lin_proj_w[:, C:2 * C], lk)
    wv_eff = jnp.einsum("lij,ljk->lik", lin_proj_w[:, 2 * C:3 * C], lv)
    wqkv_t = jnp.concatenate([wq_eff.transpose(0, 2, 1),
                              wk_eff.transpose(0, 2, 1),
                              wv_eff.transpose(0, 2, 1)], axis=2)
    bqkv = jnp.concatenate([lin_proj_b[:, 0] * scale,
                            lin_proj_b[:, 1],
                            lin_proj_b[:, 2]], axis=-1)[:, None, :]
    wout_t = lout_w.transpose(0, 2, 1)
    wffn_t = jnp.einsum("lij,ljk->lik", lfc2, lfc1).transpose(0, 2, 1)

    bf16 = jnp.bfloat16
    wlin_t = linear_w.T.astype(bf16)
    wqkv_t = wqkv_t.astype(bf16)
    wout_t = wout_t.astype(bf16)
    wffn_t = wffn_t.astype(bf16)

    xr = x.reshape(B, C, S)
    NB = _NB

    G = B // NB
    out = pl.pallas_call(
        partial(_stack_kernel, L, num_heads, NB),
        out_shape=jax.ShapeDtypeStruct((B, C, S), x.dtype),
        grid=(2, G // 2),
        in_specs=[
            pl.BlockSpec((NB, C, S), lambda c, b: (c * (G // 2) + b, 0, 0)),
            pl.BlockSpec((C, C), lambda c, b: (0, 0)),
            pl.BlockSpec((1, C), lambda c, b: (0, 0)),
            pl.BlockSpec((L, C, 3 * C), lambda c, b: (0, 0, 0)),
            pl.BlockSpec((L, 1, 3 * C), lambda c, b: (0, 0, 0)),
            pl.BlockSpec((L, C, C), lambda c, b: (0, 0, 0)),
            pl.BlockSpec((L, 1, C), lambda c, b: (0, 0, 0)),
            pl.BlockSpec((L, C, C), lambda c, b: (0, 0, 0)),
        ],
        out_specs=pl.BlockSpec((NB, C, S),
                               lambda c, b: (c * (G // 2) + b, 0, 0)),
        scratch_shapes=[
            pltpu.VMEM((NB * S, C), jnp.float32),
            pltpu.VMEM((NB * S, 3 * C), jnp.bfloat16),
        ],
        compiler_params=pltpu.CompilerParams(
            dimension_semantics=("parallel", "arbitrary")),
    )(xr, wlin_t, linear_b, wqkv_t, bqkv, wout_t, lout_b, wffn_t)

    return out.reshape(B, C, W, Hs)


# speedup vs baseline: 1.3614x; 1.0249x over previous
import math
from functools import partial

import jax
import jax.numpy as jnp
from jax import lax
from jax.experimental import pallas as pl
from jax.experimental.pallas import tpu as pltpu

_NB = 16


def _stack_kernel(L, H, NB,
                  x_ref, wlin_ref, blin_ref,
                  wqkv_ref, bqkv_ref, wout_ref, bout_ref, wffn_ref,
                  o_ref, p_sc, qkv_sc):
    C, S = x_ref.shape[1], x_ref.shape[2]
    D = C // H
    bf16 = jnp.bfloat16

    xt = jnp.concatenate([x_ref[i].T for i in range(NB)], axis=0)
    p_sc[...] = (xt
                 + jnp.dot(xt.astype(bf16), wlin_ref[...],
                           preferred_element_type=jnp.float32)
                 + blin_ref[...])

    trans_b = (((1,), (1,)), ((), ()))
    ones_col = jnp.ones((S, D), bf16)

    def layer(l, carry):
        p = p_sc[...]
        qkv_sc[...] = (jnp.dot(p.astype(bf16), wqkv_ref[l],
                               preferred_element_type=jnp.float32)
                       + bqkv_ref[l]).astype(bf16)

        rows = []
        for i in range(NB):
            r0, r1 = i * S, (i + 1) * S
            heads = []
            for h in range(H):
                q = qkv_sc[r0:r1, h * D:(h + 1) * D]
                k = qkv_sc[r0:r1, C + h * D:C + (h + 1) * D]
                v = qkv_sc[r0:r1, 2 * C + h * D:2 * C + (h + 1) * D]
                s = lax.dot_general(q, k, trans_b,
                                    preferred_element_type=jnp.float32)
                e = jnp.exp2(jnp.minimum(s, 115.0)).astype(bf16)
                o = jnp.dot(e, jnp.concatenate([v, ones_col], axis=1),
                            preferred_element_type=jnp.float32)
                heads.append(o[:, :D]
                             * pl.reciprocal(o[:, D:2 * D], approx=True))
            rows.append(jnp.concatenate(heads, axis=1))
        attn = jnp.concatenate(rows, axis=0).astype(bf16)

        x1 = (jnp.dot(attn, wout_ref[l], preferred_element_type=jnp.float32)
              + bout_ref[l] + p)
        p_sc[...] = x1 + jnp.dot(x1.astype(bf16), wffn_ref[l],
                                 preferred_element_type=jnp.float32)
        return carry

    lax.fori_loop(0, L, layer, 0)
    for i in range(NB):
        o_ref[i] = p_sc[i * S:(i + 1) * S, :].T


def kernel(x, linear_w, linear_b, lq, lk, lv, lin_proj_w, lin_proj_b,
           lout_w, lout_b, lfc1, lfc2):
    num_heads = 8
    B, C, W, Hs = x.shape
    S = W * Hs
    L = lq.shape[0]
    scale = math.log2(math.e) / math.sqrt(C // num_heads)

    wq_eff = jnp.einsum("lij,ljk->lik", lin_proj_w[:, 0:C], lq) * scale
    wk_eff = jnp.einsum("lij,ljk->lik", lin_proj_w[:, C:2 * C], lk)
    wv_eff = jnp.einsum("lij,ljk->lik", lin_proj_w[:, 2 * C:3 * C], lv)
    wqkv_t = jnp.concatenate([wq_eff.transpose(0, 2, 1),
                              wk_eff.transpose(0, 2, 1),
                              wv_eff.transpose(0, 2, 1)], axis=2)
    bqkv = jnp.concatenate([lin_proj_b[:, 0] * scale,
                            lin_proj_b[:, 1],
                            lin_proj_b[:, 2]], axis=-1)[:, None, :]
    wout_t = lout_w.transpose(0, 2, 1)
    wffn_t = jnp.einsum("lij,ljk->lik", lfc2, lfc1).transpose(0, 2, 1)

    bf16 = jnp.bfloat16
    wlin_t = linear_w.T.astype(bf16)
    wqkv_t = wqkv_t.astype(bf16)
    wout_t = wout_t.astype(bf16)
    wffn_t = wffn_t.astype(bf16)

    xr = x.reshape(B, C, S)
    NB = _NB

    G = B // NB
    out = pl.pallas_call(
        partial(_stack_kernel, L, num_heads, NB),
        out_shape=jax.ShapeDtypeStruct((B, C, S), x.dtype),
        grid=(2, G // 2),
        in_specs=[
            pl.BlockSpec((NB, C, S), lambda c, b: (c * (G // 2) + b, 0, 0)),
            pl.BlockSpec((C, C), lambda c, b: (0, 0)),
            pl.BlockSpec((1, C), lambda c, b: (0, 0)),
            pl.BlockSpec((L, C, 3 * C), lambda c, b: (0, 0, 0)),
            pl.BlockSpec((L, 1, 3 * C), lambda c, b: (0, 0, 0)),
            pl.BlockSpec((L, C, C), lambda c, b: (0, 0, 0)),
            pl.BlockSpec((L, 1, C), lambda c, b: (0, 0, 0)),
            pl.BlockSpec((L, C, C), lambda c, b: (0, 0, 0)),
        ],
        out_specs=pl.BlockSpec((NB, C, S),
                               lambda c, b: (c * (G // 2) + b, 0, 0)),
        scratch_shapes=[
            pltpu.VMEM((NB * S, C), jnp.float32),
            pltpu.VMEM((NB * S, 3 * C), jnp.bfloat16),
        ],
        compiler_params=pltpu.CompilerParams(
            dimension_semantics=("parallel", "arbitrary")),
    )(xr, wlin_t, linear_b, wqkv_t, bqkv, wout_t, lout_b, wffn_t)

    return out.reshape(B, C, W, Hs)
